# Initial kernel scaffold; baseline (speedup 1.0000x reference)
import jax
import jax.numpy as jnp
from jax.experimental import pallas as pl


def kernel(x0, x1, x2, x3, edge_attr, W1, b1, g1, beta1, W2, b2, g2, beta2, g3, beta3, eps, edge_index0, edge_index1, edge_index2, edge_index3):
    raise NotImplementedError("write your pallas kernel here")


# speedup vs baseline: 3.7255x; 3.7255x over previous
import functools

import jax
import jax.numpy as jnp
from jax import lax
from jax.experimental import pallas as pl
from jax.experimental.pallas import tpu as pltpu
from jax.experimental.pallas import tpu_sc as plsc


def _pre_body(x1r, x2r, x3r, epsr, y1r, y2r, y3r):
    y1r[...] = jnp.maximum(x1r[...], 0.0) * (1.0 + epsr[2, :])
    y2r[...] = jnp.maximum(x2r[...], 0.0) * (1.0 + epsr[3, :])
    y3r[...] = jnp.maximum(x3r[...], 0.0) * (1.0 + epsr[4, :])


def _pre_tc(x1, x2, x3, eps):
    n, d = x1.shape
    sh = jax.ShapeDtypeStruct((n, d), jnp.float32)
    return pl.pallas_call(
        _pre_body,
        out_shape=(sh, sh, sh),
    )(x1, x2, x3, eps)


def _make_sc_agg(n, d, e):
    nw = 32
    ew = e // nw
    ch = 80
    nch = ew // ch
    rpt = (n // 16) // 8 * 8
    rem = n - 15 * rpt
    rem_a = rem // 8 * 8
    rem_b = rem - rem_a
    mesh = plsc.VectorSubcoreMesh(core_axis_name="c", subcore_axis_name="s")

    @functools.partial(
        pl.kernel,
        out_type=jax.ShapeDtypeStruct((2 * n, d), jnp.float32),
        mesh=mesh,
        scratch_types=[
            pltpu.VMEM((ch,), jnp.int32),
            pltpu.VMEM((ch,), jnp.int32),
            pltpu.VMEM((ch, d), jnp.float32),
            pltpu.VMEM((ch, d), jnp.float32),
            pltpu.VMEM((d,), jnp.float32),
            pltpu.VMEM_SHARED((n, d), jnp.float32),
        ],
    )
    def sc_agg(x0h, y1h, y2h, y3h, eah, s1h,
               src0h, dst0h, src1h, dst1h, src2h, dst2h, src3h, dst3h,
               zh, outh,
               sidx, didx, grow, erow, sv, acc):
        cid = lax.axis_index("c")
        sid = lax.axis_index("s")
        wid = cid * 16 + sid
        ebase = wid * ew
        r0 = sid * rpt

        pltpu.sync_copy(zh.at[pl.ds(r0, rpt)], acc.at[pl.ds(r0, rpt)])

        @pl.when(sid == 15)
        def _():
            pltpu.sync_copy(zh.at[pl.ds(15 * rpt + rpt, rem - rpt)],
                            acc.at[pl.ds(15 * rpt + rpt, rem - rpt)])

        pltpu.sync_copy(s1h, sv)
        plsc.subcore_barrier()

        s_regs = [sv[pl.ds(16 * j, 16)] for j in range(d // 16)]

        def hop(xh, srch, dsth, hop0):
            def chunk_body(c, carry):
                base = ebase + c * ch
                pltpu.sync_copy(srch.at[pl.ds(base, ch)], sidx)
                pltpu.sync_copy(dsth.at[pl.ds(base, ch)], didx)
                pltpu.sync_copy(xh.at[sidx], grow)
                if hop0:
                    pltpu.sync_copy(eah.at[pl.ds(base, ch)], erow)

                    def row_body(r, cc):
                        for j in range(d // 16):
                            sl = pl.ds(16 * j, 16)
                            g = grow[r, sl]
                            ea = erow[r, sl]
                            grow[r, sl] = jnp.maximum(g + ea, 0.0) * s_regs[j]
                        return cc

                    lax.fori_loop(0, ch, row_body, 0)
                pltpu.sync_copy(grow, acc.at[didx], add=True)
                return carry

            lax.fori_loop(0, nch, chunk_body, 0)

        hop(x0h, src0h, dst0h, True)
        hop(y1h, src1h, dst1h, False)
        hop(y2h, src2h, dst2h, False)
        hop(y3h, src3h, dst3h, False)

        plsc.subcore_barrier()
        pltpu.sync_copy(acc.at[pl.ds(r0, rpt)],
                        outh.at[pl.ds(cid * n + r0, rpt)])

        @pl.when(sid == 15)
        def _():
            pltpu.sync_copy(acc.at[pl.ds(16 * rpt, rem - rpt)],
                            outh.at[pl.ds(cid * n + 16 * rpt, rem - rpt)])

    return sc_agg


def _bn(x, g, b):
    mu = jnp.mean(x, axis=0, keepdims=True)
    var = jnp.mean((x - mu) ** 2, axis=0, keepdims=True)
    return (x - mu) / jnp.sqrt(var + 1e-5) * g + b


def _post_body(n, x0r, pr, epsr, w1r, b1r, g1r, be1r, w2r, b2r, g2r, be2r,
               g3r, be3r, outr):
    res = (1.0 + epsr[0, :]) * x0r[...] + pr[:n, :] + pr[n:, :]
    h = lax.dot_general(res, w1r[...], (((1,), (1,)), ((), ())),
                        preferred_element_type=jnp.float32) + b1r[0, :]
    h = jnp.maximum(_bn(h, g1r[0, :], be1r[0, :]), 0.0)
    h = lax.dot_general(h, w2r[...], (((1,), (1,)), ((), ())),
                        preferred_element_type=jnp.float32) + b2r[0, :]
    h = jnp.maximum(_bn(h, g2r[0, :], be2r[0, :]), 0.0)
    outr[...] = jnp.maximum(_bn(h, g3r[0, :], be3r[0, :]), 0.0)


def _post_tc(x0, parts, eps, W1, b1, g1, beta1, W2, b2, g2, beta2, g3, beta3):
    n, d = x0.shape
    r2 = lambda v: v.reshape(1, -1)
    return pl.pallas_call(
        functools.partial(_post_body, n),
        out_shape=jax.ShapeDtypeStruct((n, d), jnp.float32),
    )(x0, parts, eps, W1, r2(b1), r2(g1), r2(beta1), W2, r2(b2), r2(g2),
      r2(beta2), r2(g3), r2(beta3))


def kernel(x0, x1, x2, x3, edge_attr, W1, b1, g1, beta1, W2, b2, g2, beta2,
           g3, beta3, eps, edge_index0, edge_index1, edge_index2, edge_index3):
    n, d = x0.shape
    e = edge_index0.shape[1]

    y1, y2, y3 = _pre_tc(x1, x2, x3, eps)
    scale1 = (1.0 + eps[1]).astype(jnp.float32)
    zeros = jnp.zeros((n, d), jnp.float32)

    sc_agg = _make_sc_agg(n, d, e)
    parts = sc_agg(x0, y1, y2, y3, edge_attr, scale1,
                   edge_index0[0], edge_index0[1],
                   edge_index1[0], edge_index1[1],
                   edge_index2[0], edge_index2[1],
                   edge_index3[0], edge_index3[1],
                   zeros)

    return _post_tc(x0, parts, eps, W1, b1, g1, beta1, W2, b2, g2, beta2,
                    g3, beta3)
